# baseline (speedup 1.0000x reference)
import functools

import jax
import jax.numpy as jnp
from jax import lax
from jax.experimental import pallas as pl
from jax.experimental.pallas import tpu as pltpu
from jax.experimental.pallas import tpu_sc as plsc

_B = 16
_TOTAL = 512
_T = 128
_D = 1024
_OUT = 256

_BU = 16
_NS = 16
_NW = 32
_RPW = _TOTAL // _NW


def _emb_body(x1_ref, x2_ref, w_ref, b_ref, o_ref):
    pooled = (jnp.sum(x1_ref[...], axis=1) +
              jnp.sum(x2_ref[...], axis=1)) * (1.0 / _T)
    emb = jnp.dot(pooled, w_ref[...],
                  preferred_element_type=jnp.float32) + b_ref[...]
    norm = jnp.sqrt(jnp.sum(emb * emb, axis=-1, keepdims=True))
    o_ref[...] = emb / (norm + 1e-8)


def _loss_body(p_ref, lab_ref, o_ref):
    m = jnp.max(p_ref[...], axis=0)
    lab = lab_ref[0, :]
    pos = jnp.where(lab > 0.0, 1.0 - m, 0.0)
    neg = jnp.where(lab < 0.0, jnp.maximum(m, 0.0), 0.0)
    o_ref[...] = jnp.reshape((jnp.sum(pos) + jnp.sum(neg)) / _B, (1, 1))


def _sc_seg_max(an_hbm, qn_hbm, cuts_hbm, out_hbm,
                an_v, q_v, cuts_v, max_v, sem_a, sem_g):
    wid = lax.axis_index("c") * _NS + lax.axis_index("s")
    base = wid * _RPW

    cp_an = pltpu.async_copy(an_hbm.at[pl.ds(base, _RPW)], an_v, sem_a)
    pltpu.sync_copy(cuts_hbm, cuts_v)

    lanes = lax.iota(jnp.int32, 16)
    j_vec = lanes + base
    cuts_vec = cuts_v[...]
    seg = jnp.zeros((16,), jnp.int32)
    for k in range(_B):
        seg = seg + jnp.where(cuts_vec[k] <= j_vec, 1, 0).astype(jnp.int32)

    pltpu.async_copy(qn_hbm.at[seg], q_v, sem_g).wait()
    cp_an.wait()

    maxvec = jnp.full((16,), -1e30, jnp.float32)
    for r in range(_RPW):
        acc = jnp.zeros((16,), jnp.float32)
        for c in range(_OUT // 16):
            sl = pl.ds(c * 16, 16)
            acc = acc + an_v[r, sl] * q_v[r, sl]
        sim = acc[0]
        for k in range(1, 16):
            sim = sim + acc[k]
        seg_r = seg[r]
        maxvec = jnp.maximum(maxvec, jnp.where(lanes == seg_r, sim, -1e30))

    max_v[...] = maxvec
    pltpu.sync_copy(max_v, out_hbm.at[wid])


@functools.cache
def _sc_seg_max_call():
    return pl.kernel(
        _sc_seg_max,
        mesh=plsc.VectorSubcoreMesh(core_axis_name="c", subcore_axis_name="s"),
        out_type=jax.ShapeDtypeStruct((_NW, 16), jnp.float32),
        scratch_types=[
            pltpu.VMEM((_RPW, _OUT), jnp.float32),
            pltpu.VMEM((_RPW, _OUT), jnp.float32),
            pltpu.VMEM((16,), jnp.int32),
            pltpu.VMEM((16,), jnp.float32),
            pltpu.SemaphoreType.DMA,
            pltpu.SemaphoreType.DMA,
        ],
    )


def kernel(features, prefix_sums, labels, W, b):
    n_utt = _B + _TOTAL
    embs_n = pl.pallas_call(
        _emb_body,
        grid=(n_utt // _BU,),
        in_specs=[
            pl.BlockSpec((_BU, _T // 2, _D), lambda i: (i, 0, 0)),
            pl.BlockSpec((_BU, _T // 2, _D), lambda i: (i, 1, 0)),
            pl.BlockSpec((_D, _OUT), lambda i: (0, 0)),
            pl.BlockSpec((1, _OUT), lambda i: (0, 0)),
        ],
        out_specs=pl.BlockSpec((_BU, _OUT), lambda i: (i, 0)),
        out_shape=jax.ShapeDtypeStruct((n_utt, _OUT), jnp.float32),
    )(features, features, W, jnp.reshape(b, (1, _OUT)))

    qn = embs_n[:_B]
    an = embs_n[_B:]
    cuts = prefix_sums[1:]

    partial = _sc_seg_max_call()(an, qn, cuts)

    loss = pl.pallas_call(
        _loss_body,
        in_specs=[
            pl.BlockSpec((_NW, 16), lambda: (0, 0)),
            pl.BlockSpec((1, _B), lambda: (0, 0)),
        ],
        out_specs=pl.BlockSpec((1, 1), lambda: (0, 0)),
        out_shape=jax.ShapeDtypeStruct((1, 1), jnp.float32),
    )(partial, jnp.reshape(labels, (1, _B)))

    return jnp.reshape(loss, ())


# speedup vs baseline: 1.0348x; 1.0348x over previous
import functools

import jax
import jax.numpy as jnp
from jax import lax
from jax.experimental import pallas as pl
from jax.experimental.pallas import tpu as pltpu
from jax.experimental.pallas import tpu_sc as plsc

_B = 16
_TOTAL = 512
_T = 128
_D = 1024
_OUT = 256

_BU = 16
_NS = 16
_NW = 32
_RPW = _TOTAL // _NW

_NSC = 160
_NTC = _B + _TOTAL - _NSC
_UPS = _NSC // _NW
_RCH = 32
_NCH = _T // _RCH


def _emb_body(x_ref, w_ref, b_ref, o_ref):
    pooled = jnp.sum(x_ref[...], axis=1) * (1.0 / _T)
    emb = jnp.dot(pooled, w_ref[...],
                  preferred_element_type=jnp.float32) + b_ref[...]
    norm = jnp.sqrt(jnp.sum(emb * emb, axis=-1, keepdims=True))
    o_ref[...] = emb / (norm + 1e-8)


def _loss_body(p_ref, lab_ref, o_ref):
    m = jnp.max(p_ref[...], axis=0)
    lab = lab_ref[0, :]
    pos = jnp.where(lab > 0.0, 1.0 - m, 0.0)
    neg = jnp.where(lab < 0.0, jnp.maximum(m, 0.0), 0.0)
    o_ref[...] = jnp.reshape((jnp.sum(pos) + jnp.sum(neg)) / _B, (1, 1))


def _sc_pool(feat_hbm, out_hbm, buf_v, acc_v, sem_a, sem_b):
    wid = lax.axis_index("c") * _NS + lax.axis_index("s")
    sems = (sem_a, sem_b)

    def do_utt(t, _):
        u = _NTC + wid * _UPS + t

        def stage(chunk):
            return pltpu.async_copy(
                feat_hbm.at[u, pl.ds(chunk * _RCH, _RCH)],
                buf_v.at[chunk % 2], sems[chunk % 2])

        cp = stage(0)
        for chunk in range(_NCH):
            cp.wait()
            if chunk + 1 < _NCH:
                cp = stage(chunk + 1)
            for cg in range(4):
                def body(r, accs):
                    return tuple(
                        accs[k] + buf_v[chunk % 2, r,
                                        pl.ds(cg * 256 + k * 16, 16)]
                        for k in range(16))
                accs = lax.fori_loop(
                    0, _RCH, body,
                    tuple(jnp.zeros((16,), jnp.float32) for _ in range(16)))
                for k in range(16):
                    sl = pl.ds(cg * 256 + k * 16, 16)
                    if chunk == 0:
                        acc_v[sl] = accs[k]
                    else:
                        acc_v[sl] = acc_v[sl] + accs[k]
        pltpu.sync_copy(acc_v, out_hbm.at[wid * _UPS + t])
        return 0

    lax.fori_loop(0, _UPS, do_utt, 0)


@functools.cache
def _sc_pool_call():
    return pl.kernel(
        _sc_pool,
        mesh=plsc.VectorSubcoreMesh(core_axis_name="c", subcore_axis_name="s"),
        out_type=jax.ShapeDtypeStruct((_NSC, _D), jnp.float32),
        scratch_types=[
            pltpu.VMEM((2, _RCH, _D), jnp.float32),
            pltpu.VMEM((_D,), jnp.float32),
            pltpu.SemaphoreType.DMA,
            pltpu.SemaphoreType.DMA,
        ],
    )


def _proj_body(p_ref, w_ref, b_ref, o_ref):
    emb = jnp.dot(p_ref[...] * (1.0 / _T), w_ref[...],
                  preferred_element_type=jnp.float32) + b_ref[...]
    norm = jnp.sqrt(jnp.sum(emb * emb, axis=-1, keepdims=True))
    o_ref[...] = emb / (norm + 1e-8)


def _sc_seg_max(an_hbm, qn_hbm, cuts_hbm, out_hbm,
                an_v, q_v, cuts_v, max_v, sem_a, sem_g):
    wid = lax.axis_index("c") * _NS + lax.axis_index("s")
    base = wid * _RPW

    cp_an = pltpu.async_copy(an_hbm.at[pl.ds(base, _RPW)], an_v, sem_a)
    pltpu.sync_copy(cuts_hbm, cuts_v)

    lanes = lax.iota(jnp.int32, 16)
    j_vec = lanes + base
    cuts_vec = cuts_v[...]
    seg = jnp.zeros((16,), jnp.int32)
    for k in range(_B):
        seg = seg + jnp.where(cuts_vec[k] <= j_vec, 1, 0).astype(jnp.int32)

    pltpu.async_copy(qn_hbm.at[seg], q_v, sem_g).wait()
    cp_an.wait()

    maxvec = jnp.full((16,), -1e30, jnp.float32)
    for r in range(_RPW):
        acc = jnp.zeros((16,), jnp.float32)
        for c in range(_OUT // 16):
            sl = pl.ds(c * 16, 16)
            acc = acc + an_v[r, sl] * q_v[r, sl]
        sim = acc[0]
        for k in range(1, 16):
            sim = sim + acc[k]
        seg_r = seg[r]
        maxvec = jnp.maximum(maxvec, jnp.where(lanes == seg_r, sim, -1e30))

    max_v[...] = maxvec
    pltpu.sync_copy(max_v, out_hbm.at[wid])


@functools.cache
def _sc_seg_max_call():
    return pl.kernel(
        _sc_seg_max,
        mesh=plsc.VectorSubcoreMesh(core_axis_name="c", subcore_axis_name="s"),
        out_type=jax.ShapeDtypeStruct((_NW, 16), jnp.float32),
        scratch_types=[
            pltpu.VMEM((_RPW, _OUT), jnp.float32),
            pltpu.VMEM((_RPW, _OUT), jnp.float32),
            pltpu.VMEM((16,), jnp.int32),
            pltpu.VMEM((16,), jnp.float32),
            pltpu.SemaphoreType.DMA,
            pltpu.SemaphoreType.DMA,
        ],
    )


def kernel(features, prefix_sums, labels, W, b):
    b2 = jnp.reshape(b, (1, _OUT))

    pooled_sc = _sc_pool_call()(features)

    embs_a = pl.pallas_call(
        _emb_body,
        grid=(_NTC // _BU,),
        in_specs=[
            pl.BlockSpec((_BU, _T, _D), lambda i: (i, 0, 0)),
            pl.BlockSpec((_D, _OUT), lambda i: (0, 0)),
            pl.BlockSpec((1, _OUT), lambda i: (0, 0)),
        ],
        out_specs=pl.BlockSpec((_BU, _OUT), lambda i: (i, 0)),
        out_shape=jax.ShapeDtypeStruct((_NTC, _OUT), jnp.float32),
    )(features, W, b2)

    embs_b = pl.pallas_call(
        _proj_body,
        in_specs=[
            pl.BlockSpec((_NSC, _D), lambda: (0, 0)),
            pl.BlockSpec((_D, _OUT), lambda: (0, 0)),
            pl.BlockSpec((1, _OUT), lambda: (0, 0)),
        ],
        out_specs=pl.BlockSpec((_NSC, _OUT), lambda: (0, 0)),
        out_shape=jax.ShapeDtypeStruct((_NSC, _OUT), jnp.float32),
    )(pooled_sc, W, b2)

    embs_n = jnp.concatenate([embs_a, embs_b], axis=0)

    qn = embs_n[:_B]
    an = embs_n[_B:]
    cuts = prefix_sums[1:]

    partial = _sc_seg_max_call()(an, qn, cuts)

    loss = pl.pallas_call(
        _loss_body,
        in_specs=[
            pl.BlockSpec((_NW, 16), lambda: (0, 0)),
            pl.BlockSpec((1, _B), lambda: (0, 0)),
        ],
        out_specs=pl.BlockSpec((1, 1), lambda: (0, 0)),
        out_shape=jax.ShapeDtypeStruct((1, 1), jnp.float32),
    )(partial, jnp.reshape(labels, (1, _B)))

    return jnp.reshape(loss, ())


# speedup vs baseline: 1.0718x; 1.0358x over previous
import functools

import jax
import jax.numpy as jnp
from jax import lax
from jax.experimental import pallas as pl
from jax.experimental.pallas import tpu as pltpu
from jax.experimental.pallas import tpu_sc as plsc

_B = 16
_TOTAL = 512
_T = 128
_D = 1024
_OUT = 256

_BU = 16
_NS = 16
_NW = 32
_RPW = _TOTAL // _NW


def _emb_body(x_ref, w_ref, b_ref, o_ref):
    pooled = jnp.sum(x_ref[...], axis=1) * (1.0 / _T)
    emb = jnp.dot(pooled, w_ref[...],
                  preferred_element_type=jnp.float32) + b_ref[...]
    norm = jnp.sqrt(jnp.sum(emb * emb, axis=-1, keepdims=True))
    o_ref[...] = emb / (norm + 1e-8)


def _loss_body(p_ref, lab_ref, o_ref):
    m = jnp.max(p_ref[...], axis=0)
    lab = lab_ref[0, :]
    pos = jnp.where(lab > 0.0, 1.0 - m, 0.0)
    neg = jnp.where(lab < 0.0, jnp.maximum(m, 0.0), 0.0)
    o_ref[...] = jnp.reshape((jnp.sum(pos) + jnp.sum(neg)) / _B, (1, 1))


def _sc_seg_max(an_hbm, qn_hbm, cuts_hbm, out_hbm,
                an_v, q_v, cuts_v, max_v, sem_a, sem_g):
    wid = lax.axis_index("c") * _NS + lax.axis_index("s")
    base = wid * _RPW

    cp_an = pltpu.async_copy(an_hbm.at[pl.ds(base, _RPW)], an_v, sem_a)
    pltpu.sync_copy(cuts_hbm, cuts_v)

    lanes = lax.iota(jnp.int32, 16)
    j_vec = lanes + base
    cuts_vec = cuts_v[...]
    seg = jnp.zeros((16,), jnp.int32)
    for k in range(_B):
        seg = seg + jnp.where(cuts_vec[k] <= j_vec, 1, 0).astype(jnp.int32)

    pltpu.async_copy(qn_hbm.at[seg], q_v, sem_g).wait()
    cp_an.wait()

    def shuffle(v, idx):
        return lax.gather(
            v, jnp.reshape(idx, (16, 1)),
            dimension_numbers=lax.GatherDimensionNumbers(
                offset_dims=(), collapsed_slice_dims=(0,),
                start_index_map=(0,)),
            slice_sizes=(1,),
            mode=lax.GatherScatterMode.PROMISE_IN_BOUNDS)

    maxvec = jnp.full((16,), -1e30, jnp.float32)
    for r in range(_RPW):
        acc = jnp.zeros((16,), jnp.float32)
        for c in range(_OUT // 16):
            sl = pl.ds(c * 16, 16)
            acc = acc + an_v[r, sl] * q_v[r, sl]
        for sh in (8, 4, 2, 1):
            acc = acc + shuffle(acc, lax.bitwise_xor(lanes, sh))
        seg_r = shuffle(seg, jnp.full((16,), r, jnp.int32))
        maxvec = jnp.maximum(maxvec, jnp.where(lanes == seg_r, acc, -1e30))

    max_v[...] = maxvec
    pltpu.sync_copy(max_v, out_hbm.at[wid])


@functools.cache
def _sc_seg_max_call():
    return pl.kernel(
        _sc_seg_max,
        mesh=plsc.VectorSubcoreMesh(core_axis_name="c", subcore_axis_name="s"),
        out_type=jax.ShapeDtypeStruct((_NW, 16), jnp.float32),
        scratch_types=[
            pltpu.VMEM((_RPW, _OUT), jnp.float32),
            pltpu.VMEM((_RPW, _OUT), jnp.float32),
            pltpu.VMEM((16,), jnp.int32),
            pltpu.VMEM((16,), jnp.float32),
            pltpu.SemaphoreType.DMA,
            pltpu.SemaphoreType.DMA,
        ],
    )


def kernel(features, prefix_sums, labels, W, b):
    n_utt = _B + _TOTAL
    embs_n = pl.pallas_call(
        _emb_body,
        grid=(n_utt // _BU,),
        in_specs=[
            pl.BlockSpec((_BU, _T, _D), lambda i: (i, 0, 0)),
            pl.BlockSpec((_D, _OUT), lambda i: (0, 0)),
            pl.BlockSpec((1, _OUT), lambda i: (0, 0)),
        ],
        out_specs=pl.BlockSpec((_BU, _OUT), lambda i: (i, 0)),
        out_shape=jax.ShapeDtypeStruct((n_utt, _OUT), jnp.float32),
    )(features, W, jnp.reshape(b, (1, _OUT)))

    qn = embs_n[:_B]
    an = embs_n[_B:]
    cuts = prefix_sums[1:]

    partial = _sc_seg_max_call()(an, qn, cuts)

    loss = pl.pallas_call(
        _loss_body,
        in_specs=[
            pl.BlockSpec((_NW, 16), lambda: (0, 0)),
            pl.BlockSpec((1, _B), lambda: (0, 0)),
        ],
        out_specs=pl.BlockSpec((1, 1), lambda: (0, 0)),
        out_shape=jax.ShapeDtypeStruct((1, 1), jnp.float32),
    )(partial, jnp.reshape(labels, (1, _B)))

    return jnp.reshape(loss, ())


# speedup vs baseline: 1.0968x; 1.0233x over previous
import functools

import jax
import jax.numpy as jnp
from jax import lax
from jax.experimental import pallas as pl
from jax.experimental.pallas import tpu as pltpu
from jax.experimental.pallas import tpu_sc as plsc

_B = 16
_TOTAL = 512
_T = 128
_D = 1024
_OUT = 256

_BU = 16
_NS = 16
_NW = 32
_RPW = _TOTAL // _NW


def _emb_body(x_ref, w_ref, b_ref, o_ref):
    pooled = jnp.sum(x_ref[...], axis=1) * (1.0 / _T)
    emb = jnp.dot(pooled, w_ref[...],
                  preferred_element_type=jnp.float32) + b_ref[...]
    norm = jnp.sqrt(jnp.sum(emb * emb, axis=-1, keepdims=True))
    o_ref[...] = emb / (norm + 1e-8)


def _loss_body(p_ref, lab_ref, o_ref):
    m = jnp.max(p_ref[...], axis=0)
    lab = lab_ref[0, :]
    pos = jnp.where(lab > 0.0, 1.0 - m, 0.0)
    neg = jnp.where(lab < 0.0, jnp.maximum(m, 0.0), 0.0)
    o_ref[...] = jnp.reshape((jnp.sum(pos) + jnp.sum(neg)) / _B, (1, 1))


def _sc_seg_max(embs_hbm, cuts_hbm, out_hbm,
                an_v, q_v, cuts_v, max_v, sem_a, sem_g):
    wid = lax.axis_index("c") * _NS + lax.axis_index("s")
    base = wid * _RPW

    cp_an = pltpu.async_copy(embs_hbm.at[pl.ds(_B + base, _RPW)], an_v, sem_a)
    pltpu.sync_copy(cuts_hbm, cuts_v)

    lanes = lax.iota(jnp.int32, 16)
    j_vec = lanes + base
    cuts_vec = cuts_v[...]
    seg = jnp.zeros((16,), jnp.int32)
    for k in range(_B):
        seg = seg + jnp.where(cuts_vec[k] <= j_vec, 1, 0).astype(jnp.int32)

    pltpu.async_copy(embs_hbm.at[seg], q_v, sem_g).wait()
    cp_an.wait()

    def shuffle(v, idx):
        return lax.gather(
            v, jnp.reshape(idx, (16, 1)),
            dimension_numbers=lax.GatherDimensionNumbers(
                offset_dims=(), collapsed_slice_dims=(0,),
                start_index_map=(0,)),
            slice_sizes=(1,),
            mode=lax.GatherScatterMode.PROMISE_IN_BOUNDS)

    maxvec = jnp.full((16,), -1e30, jnp.float32)
    for r in range(_RPW):
        acc = jnp.zeros((16,), jnp.float32)
        for c in range(_OUT // 16):
            sl = pl.ds(c * 16, 16)
            acc = acc + an_v[r, sl] * q_v[r, sl]
        for sh in (8, 4, 2, 1):
            acc = acc + shuffle(acc, lax.bitwise_xor(lanes, sh))
        seg_r = shuffle(seg, jnp.full((16,), r, jnp.int32))
        maxvec = jnp.maximum(maxvec, jnp.where(lanes == seg_r, acc, -1e30))

    max_v[...] = maxvec
    pltpu.sync_copy(max_v, out_hbm.at[wid])


@functools.cache
def _sc_seg_max_call():
    return pl.kernel(
        _sc_seg_max,
        mesh=plsc.VectorSubcoreMesh(core_axis_name="c", subcore_axis_name="s"),
        out_type=jax.ShapeDtypeStruct((_NW, 16), jnp.float32),
        scratch_types=[
            pltpu.VMEM((_RPW, _OUT), jnp.float32),
            pltpu.VMEM((_RPW, _OUT), jnp.float32),
            pltpu.VMEM((16,), jnp.int32),
            pltpu.VMEM((16,), jnp.float32),
            pltpu.SemaphoreType.DMA,
            pltpu.SemaphoreType.DMA,
        ],
    )


def kernel(features, prefix_sums, labels, W, b):
    n_utt = _B + _TOTAL
    embs_n = pl.pallas_call(
        _emb_body,
        grid=(n_utt // _BU,),
        in_specs=[
            pl.BlockSpec((_BU, _T, _D), lambda i: (i, 0, 0)),
            pl.BlockSpec((_D, _OUT), lambda i: (0, 0)),
            pl.BlockSpec((1, _OUT), lambda i: (0, 0)),
        ],
        out_specs=pl.BlockSpec((_BU, _OUT), lambda i: (i, 0)),
        out_shape=jax.ShapeDtypeStruct((n_utt, _OUT), jnp.float32),
    )(features, W, jnp.reshape(b, (1, _OUT)))

    cuts = prefix_sums[1:]

    partial = _sc_seg_max_call()(embs_n, cuts)

    loss = pl.pallas_call(
        _loss_body,
        in_specs=[
            pl.BlockSpec((_NW, 16), lambda: (0, 0)),
            pl.BlockSpec((1, _B), lambda: (0, 0)),
        ],
        out_specs=pl.BlockSpec((1, 1), lambda: (0, 0)),
        out_shape=jax.ShapeDtypeStruct((1, 1), jnp.float32),
    )(partial, jnp.reshape(labels, (1, _B)))

    return jnp.reshape(loss, ())
